# baseline (speedup 1.0000x reference)
import functools

import jax
import jax.numpy as jnp
from jax import lax
from jax.experimental import pallas as pl
from jax.experimental.pallas import tpu as pltpu
from jax.experimental.pallas import tpu_sc as plsc

SEQ = 200
EMBED = 64
LANES = 16
NUM_WORKERS = 32
CHUNK = 800
GATHER_W = 200
EMBED_SCALE = 8.0
NDIM = EMBED // LANES


def _body(idx_hbm, tok_hbm, pos_hbm, out_hbm, idx_v, rows_v, out_v, pos_v, sem):
    c = lax.axis_index("c")
    s = lax.axis_index("s")
    wid = s * 2 + c
    n_rows = out_hbm.shape[0] // EMBED
    rows_per_worker = n_rows // NUM_WORKERS
    chunks_per_worker = rows_per_worker // CHUNK

    pltpu.sync_copy(pos_hbm, pos_v)

    def chunk_body(ci, _):
        base = wid * rows_per_worker + ci * CHUNK

        pltpu.sync_copy(idx_hbm.at[pl.ds(base, CHUNK)], idx_v)

        copies = []
        for j in range(CHUNK // GATHER_W):
            copies.append(
                pltpu.async_copy(
                    tok_hbm.at[idx_v.at[pl.ds(j * GATHER_W, GATHER_W)]],
                    rows_v.at[pl.ds(j * GATHER_W, GATHER_W)],
                    sem,
                )
            )
        for cp in copies:
            cp.wait()

        def pos_body(p, _):
            pv = [pos_v[p, pl.ds(d * LANES, LANES)] for d in range(NDIM)]
            for jb in range(CHUNK // SEQ):
                r = jb * SEQ + p
                for d in range(NDIM):
                    out_v[pl.ds(r * EMBED + d * LANES, LANES)] = (
                        rows_v[r, pl.ds(d * LANES, LANES)] * EMBED_SCALE
                        + pv[d]
                    )
            return _

        lax.fori_loop(0, SEQ, pos_body, None)

        pltpu.sync_copy(out_v, out_hbm.at[pl.ds(base * EMBED, CHUNK * EMBED)])
        return _

    lax.fori_loop(0, chunks_per_worker, chunk_body, None)


def kernel(inputs, token_table, position_table):
    batch, seq = inputs.shape
    n_rows = batch * seq
    zf = position_table[0, 0] * 0.0
    zi = zf.astype(jnp.int32)
    idx = inputs.reshape(n_rows) + zi * jnp.arange(n_rows, dtype=jnp.int32)

    mesh = plsc.VectorSubcoreMesh(core_axis_name="c", subcore_axis_name="s")
    k = functools.partial(
        pl.kernel,
        mesh=mesh,
        out_type=jax.ShapeDtypeStruct((n_rows * EMBED,), jnp.float32),
        scratch_types=[
            pltpu.VMEM((CHUNK,), jnp.int32),
            pltpu.VMEM((CHUNK, EMBED), jnp.float32),
            pltpu.VMEM((CHUNK * EMBED,), jnp.float32),
            pltpu.VMEM((SEQ, EMBED), jnp.float32),
            pltpu.SemaphoreType.DMA,
        ],
        compiler_params=pltpu.CompilerParams(use_tc_tiling_on_sc=False),
    )(_body)

    flat = k(idx, token_table, position_table)
    out2 = flat.reshape(n_rows, EMBED) + zf * jnp.arange(
        EMBED, dtype=jnp.float32
    )
    return out2.reshape(batch, seq, EMBED)


# speedup vs baseline: 1.1980x; 1.1980x over previous
import functools

import jax
import jax.numpy as jnp
from jax import lax
from jax.experimental import pallas as pl
from jax.experimental.pallas import tpu as pltpu
from jax.experimental.pallas import tpu_sc as plsc

SEQ = 200
EMBED = 64
LANES = 16
NUM_WORKERS = 32
ROWS_PER_CHUNK = 4
EMBED_SCALE = 8.0


def _body(idx_hbm, tok_hbm, pos_hbm, out_hbm, idx_v, rows_v, pos_v, sem):
    c = lax.axis_index("c")
    s = lax.axis_index("s")
    wid = s * 2 + c
    batch = out_hbm.shape[0]
    rows_per_worker = batch // NUM_WORKERS
    chunks_per_worker = rows_per_worker // ROWS_PER_CHUNK

    pltpu.sync_copy(pos_hbm, pos_v)

    def chunk_body(ci, _):
        b0 = wid * rows_per_worker + ci * ROWS_PER_CHUNK

        pltpu.sync_copy(idx_hbm.at[pl.ds(b0, ROWS_PER_CHUNK)], idx_v)

        copies = []
        for jb in range(ROWS_PER_CHUNK):
            copies.append(
                pltpu.async_copy(
                    tok_hbm.at[idx_v.at[jb]],
                    rows_v.at[jb],
                    sem,
                )
            )
        for cp in copies:
            cp.wait()

        def pos_body(p, _):
            pv = [pos_v[p, pl.ds(d * LANES, LANES)] for d in range(EMBED // LANES)]
            for jb in range(ROWS_PER_CHUNK):
                for d in range(EMBED // LANES):
                    sl = pl.ds(d * LANES, LANES)
                    rows_v[jb, p, sl] = rows_v[jb, p, sl] * EMBED_SCALE + pv[d]
            return _

        lax.fori_loop(0, SEQ, pos_body, None)

        pltpu.sync_copy(rows_v, out_hbm.at[pl.ds(b0, ROWS_PER_CHUNK)])
        return _

    lax.fori_loop(0, chunks_per_worker, chunk_body, None)


def kernel(inputs, token_table, position_table):
    batch, seq = inputs.shape

    mesh = plsc.VectorSubcoreMesh(core_axis_name="c", subcore_axis_name="s")
    k = functools.partial(
        pl.kernel,
        mesh=mesh,
        out_type=jax.ShapeDtypeStruct((batch, seq, EMBED), jnp.float32),
        scratch_types=[
            pltpu.VMEM((ROWS_PER_CHUNK, SEQ), jnp.int32),
            pltpu.VMEM((ROWS_PER_CHUNK, SEQ, EMBED), jnp.float32),
            pltpu.VMEM((SEQ, EMBED), jnp.float32),
            pltpu.SemaphoreType.DMA,
        ],
        compiler_params=pltpu.CompilerParams(use_tc_tiling_on_sc=False),
    )(_body)

    return k(inputs, token_table, position_table)
